# baseline (speedup 1.0000x reference)
import jax
import jax.numpy as jnp
from jax import lax
from jax.experimental import pallas as pl
from jax.experimental.pallas import tpu as pltpu
from jax.experimental.pallas import tpu_sc as plsc

_SC_PARAMS = pltpu.CompilerParams(needs_layout_passes=False,
                                  use_tc_tiling_on_sc=False)

N_NODES = 10000
N_EDGES = 320000
NODE_LEN = 128
GRAPH_LEN = 256
MPNN_STEPS = 3

_info = plsc.get_sparse_core_info()
NC, NS, L = _info.num_cores, _info.num_subcores, _info.num_lanes
F = NODE_LEN // NC
CHUNK = 128
N_CHUNKS = N_EDGES // CHUNK
RING = 3
BASE_CH = N_CHUNKS // NS
EXTRA = N_CHUNKS - BASE_CH * NS
ROWS_PT = N_NODES // NS
ZROWS = 25


def _prop_body(ei_hbm, x_hbm, x2_hbm, ha_hbm, hb_hbm, h3_hbm,
               src2_v, dst2_v, bufs_v, zbuf_v, acc_sh, gsem, ssem):
    cid = lax.axis_index("c")
    sid = lax.axis_index("s")
    c0 = cid * F
    r0 = sid * ROWS_PT

    ch0 = jnp.where(sid < EXTRA, sid * (BASE_CH + 1),
                    EXTRA + sid * BASE_CH)

    i1 = pltpu.async_copy(ei_hbm.at[0, pl.ds(ch0, BASE_CH)],
                          src2_v.at[pl.ds(0, BASE_CH)], gsem)
    i2 = pltpu.async_copy(ei_hbm.at[1, pl.ds(ch0, BASE_CH)],
                          dst2_v.at[pl.ds(0, BASE_CH)], gsem)
    i1.wait()
    i2.wait()

    @pl.when(sid < EXTRA)
    def _():
        j1 = pltpu.async_copy(ei_hbm.at[0, pl.ds(ch0 + BASE_CH, 1)],
                              src2_v.at[pl.ds(BASE_CH, 1)], gsem)
        j2 = pltpu.async_copy(ei_hbm.at[1, pl.ds(ch0 + BASE_CH, 1)],
                              dst2_v.at[pl.ds(BASE_CH, 1)], gsem)
        j1.wait()
        j2.wait()

    for k in range(ROWS_PT // ZROWS):
        rr = r0 + k * ZROWS
        pltpu.sync_copy(x_hbm.at[pl.ds(rr, ZROWS), pl.ds(c0, F)], zbuf_v)
        pltpu.sync_copy(zbuf_v, x2_hbm.at[cid, pl.ds(rr, ZROWS), :])

    zero16 = jnp.zeros((L,), jnp.float32)

    @plsc.parallel_loop(0, ZROWS, unroll=4)
    def _(i):
        for cb in range(F // L):
            zbuf_v[i, pl.ds(cb * L, L)] = zero16

    def _zero_acc():
        for k in range(ROWS_PT // ZROWS):
            pltpu.sync_copy(zbuf_v, acc_sh.at[pl.ds(r0 + k * ZROWS, ZROWS), :])

    _zero_acc()
    plsc.subcore_barrier()

    hins = (x2_hbm, ha_hbm, hb_hbm)
    houts = (ha_hbm, hb_hbm, None)

    def _gather_wait(hin):
        pltpu.make_async_copy(hin.at[pl.ds(0, CHUNK), :],
                              bufs_v.at[0], gsem).wait()

    def _scatter_drain():
        pltpu.make_async_copy(bufs_v.at[0],
                              acc_sh.at[pl.ds(0, CHUNK), :], ssem).wait()

    for rnd in range(MPNN_STEPS):
        hin = hins[rnd].at[cid]
        for k in range(RING):
            pltpu.async_copy(hin.at[src2_v.at[k]], bufs_v.at[k % (2 * RING)],
                             gsem)

        def step(i, _):
            slot = lax.rem(i, 2 * RING)

            @pl.when(i >= RING)
            def _():
                _scatter_drain()

            @pl.when(i + RING < BASE_CH)
            def _():
                pltpu.async_copy(hin.at[src2_v.at[i + RING]],
                                 bufs_v.at[lax.rem(i + RING, 2 * RING)], gsem)

            _gather_wait(hin)
            pltpu.async_copy(bufs_v.at[slot], acc_sh.at[dst2_v.at[i]], ssem,
                             add=True)
            return 0

        lax.fori_loop(0, BASE_CH, step, 0)
        for _ in range(RING):
            _scatter_drain()

        @pl.when(sid < EXTRA)
        def _():
            pltpu.async_copy(hin.at[src2_v.at[BASE_CH]],
                             bufs_v.at[0], gsem).wait()
            pltpu.sync_copy(bufs_v.at[0], acc_sh.at[dst2_v.at[BASE_CH]],
                            add=True)

        plsc.subcore_barrier()
        if rnd < MPNN_STEPS - 1:
            pltpu.sync_copy(acc_sh.at[pl.ds(r0, ROWS_PT), :],
                            houts[rnd].at[cid, pl.ds(r0, ROWS_PT), :])
            _zero_acc()
        else:
            pltpu.sync_copy(acc_sh.at[pl.ds(r0, ROWS_PT), :],
                            h3_hbm.at[pl.ds(r0, ROWS_PT), pl.ds(c0, F)])
        plsc.subcore_barrier()


def _prop(ei3, x):
    mesh = plsc.VectorSubcoreMesh(core_axis_name="c", subcore_axis_name="s")
    f = pl.kernel(
        _prop_body,
        mesh=mesh,
        compiler_params=_SC_PARAMS,
        out_type=(
            jax.ShapeDtypeStruct((NC, N_NODES, F), jnp.float32),
            jax.ShapeDtypeStruct((NC, N_NODES, F), jnp.float32),
            jax.ShapeDtypeStruct((NC, N_NODES, F), jnp.float32),
            jax.ShapeDtypeStruct((N_NODES, NODE_LEN), jnp.float32),
        ),
        scratch_types=[
            pltpu.VMEM((BASE_CH + 1, CHUNK), jnp.int32),
            pltpu.VMEM((BASE_CH + 1, CHUNK), jnp.int32),
            pltpu.VMEM((2 * RING, CHUNK, F), jnp.float32),
            pltpu.VMEM((ZROWS, F), jnp.float32),
            pltpu.VMEM_SHARED((N_NODES, F), jnp.float32),
            pltpu.SemaphoreType.DMA,
            pltpu.SemaphoreType.DMA,
        ],
    )
    return f(ei3, x)


def _k6_body(h3_ref, wp_ref, wr_ref, b_ref, out_ref, acc_ref):
    i = pl.program_id(0)

    @pl.when(i == 0)
    def _():
        acc_ref[...] = jnp.zeros_like(acc_ref)

    hp = lax.dot_general(h3_ref[...], wp_ref[:, :NODE_LEN],
                         (((1,), (1,)), ((), ())),
                         preferred_element_type=jnp.float32)
    acc_ref[...] += jnp.sum(hp, axis=0, keepdims=True)

    @pl.when(i == pl.num_programs(0) - 1)
    def _():
        g = acc_ref[...] * (1.0 / N_NODES)
        out_ref[...] = (jnp.sum(g * wr_ref[...]) + b_ref[0, 0]).reshape(1, 1)


def _k6(h3, W_pool, W_read, b2):
    blk = 1000
    grid = N_NODES // blk
    return pl.pallas_call(
        _k6_body,
        grid=(grid,),
        in_specs=[
            pl.BlockSpec((blk, NODE_LEN), lambda i: (i, 0)),
            pl.BlockSpec(W_pool.shape, lambda i: (0, 0)),
            pl.BlockSpec(W_read.shape, lambda i: (0, 0)),
            pl.BlockSpec((1, 1), lambda i: (0, 0)),
        ],
        out_specs=pl.BlockSpec((1, 1), lambda i: (0, 0)),
        out_shape=jax.ShapeDtypeStruct((1, 1), jnp.float32),
        scratch_shapes=[pltpu.VMEM((1, GRAPH_LEN), jnp.float32)],
    )(h3, W_pool, W_read, b2)


def kernel(x, edge_index, W_pool, W_read, b_read):
    ei3 = edge_index.astype(jnp.int32).reshape(2, N_CHUNKS, CHUNK)
    _, _, _, h3 = _prop(ei3, x)
    out = _k6(h3, W_pool, W_read, b_read.reshape(1, 1))
    return out.reshape(1)


# speedup vs baseline: 1.0545x; 1.0545x over previous
import jax
import jax.numpy as jnp
from jax import lax
from jax.experimental import pallas as pl
from jax.experimental.pallas import tpu as pltpu
from jax.experimental.pallas import tpu_sc as plsc

_SC_PARAMS = pltpu.CompilerParams(needs_layout_passes=False,
                                  use_tc_tiling_on_sc=False)

N_NODES = 10000
N_EDGES = 320000
NODE_LEN = 128
GRAPH_LEN = 256
MPNN_STEPS = 3

_info = plsc.get_sparse_core_info()
NC, NS, L = _info.num_cores, _info.num_subcores, _info.num_lanes
F = NODE_LEN // NC
CHUNK = 128
N_CHUNKS = N_EDGES // CHUNK
RING = 4
BASE_CH = N_CHUNKS // NS
EXTRA = N_CHUNKS - BASE_CH * NS
ROWS_PT = N_NODES // NS
ZROWS = 125


def _prop_body(ei_hbm, x_hbm, x2_hbm, ha_hbm, hb_hbm, h3_hbm,
               src2_v, dst2_v, bufs_v, zbuf_v, acc_sh, gsem, ssem):
    cid = lax.axis_index("c")
    sid = lax.axis_index("s")
    c0 = cid * F
    r0 = sid * ROWS_PT

    ch0 = jnp.where(sid < EXTRA, sid * (BASE_CH + 1),
                    EXTRA + sid * BASE_CH)

    i1 = pltpu.async_copy(ei_hbm.at[0, pl.ds(ch0, BASE_CH)],
                          src2_v.at[pl.ds(0, BASE_CH)], gsem)
    i2 = pltpu.async_copy(ei_hbm.at[1, pl.ds(ch0, BASE_CH)],
                          dst2_v.at[pl.ds(0, BASE_CH)], gsem)
    i1.wait()
    i2.wait()

    @pl.when(sid < EXTRA)
    def _():
        j1 = pltpu.async_copy(ei_hbm.at[0, pl.ds(ch0 + BASE_CH, 1)],
                              src2_v.at[pl.ds(BASE_CH, 1)], gsem)
        j2 = pltpu.async_copy(ei_hbm.at[1, pl.ds(ch0 + BASE_CH, 1)],
                              dst2_v.at[pl.ds(BASE_CH, 1)], gsem)
        j1.wait()
        j2.wait()

    for k in range(ROWS_PT // ZROWS):
        rr = r0 + k * ZROWS
        pltpu.sync_copy(x_hbm.at[pl.ds(rr, ZROWS), pl.ds(c0, F)], zbuf_v)
        pltpu.sync_copy(zbuf_v, x2_hbm.at[cid, pl.ds(rr, ZROWS), :])

    zero16 = jnp.zeros((L,), jnp.float32)

    @plsc.parallel_loop(0, ZROWS, unroll=4)
    def _(i):
        for cb in range(F // L):
            zbuf_v[i, pl.ds(cb * L, L)] = zero16

    def _zero_acc():
        for k in range(ROWS_PT // ZROWS):
            pltpu.sync_copy(zbuf_v, acc_sh.at[pl.ds(r0 + k * ZROWS, ZROWS), :])

    _zero_acc()
    plsc.subcore_barrier()

    hins = (x2_hbm, ha_hbm, hb_hbm)
    houts = (ha_hbm, hb_hbm, None)

    for rnd in range(MPNN_STEPS):
        hin = hins[rnd].at[cid]
        for k in range(RING):
            pltpu.async_copy(hin.at[src2_v.at[k]], bufs_v.at[k], gsem)

        def blk(b, _):
            for k in range(RING):
                j = b * RING + k
                pltpu.make_async_copy(
                    hin.at[pl.ds(0, CHUNK), :], bufs_v.at[k], gsem).wait()
                pltpu.sync_copy(bufs_v.at[k], acc_sh.at[dst2_v.at[j]],
                                add=True)

                @pl.when(j + RING < BASE_CH)
                def _():
                    pltpu.async_copy(hin.at[src2_v.at[j + RING]],
                                     bufs_v.at[k], gsem)
            return 0

        lax.fori_loop(0, BASE_CH // RING, blk, 0)

        @pl.when(sid < EXTRA)
        def _():
            pltpu.async_copy(hin.at[src2_v.at[BASE_CH]],
                             bufs_v.at[0], gsem).wait()
            pltpu.sync_copy(bufs_v.at[0], acc_sh.at[dst2_v.at[BASE_CH]],
                            add=True)

        plsc.subcore_barrier()
        if rnd < MPNN_STEPS - 1:
            pltpu.sync_copy(acc_sh.at[pl.ds(r0, ROWS_PT), :],
                            houts[rnd].at[cid, pl.ds(r0, ROWS_PT), :])
            _zero_acc()
        else:
            pltpu.sync_copy(acc_sh.at[pl.ds(r0, ROWS_PT), :],
                            h3_hbm.at[pl.ds(r0, ROWS_PT), pl.ds(c0, F)])
        plsc.subcore_barrier()


def _prop(ei3, x):
    mesh = plsc.VectorSubcoreMesh(core_axis_name="c", subcore_axis_name="s")
    f = pl.kernel(
        _prop_body,
        mesh=mesh,
        compiler_params=_SC_PARAMS,
        out_type=(
            jax.ShapeDtypeStruct((NC, N_NODES, F), jnp.float32),
            jax.ShapeDtypeStruct((NC, N_NODES, F), jnp.float32),
            jax.ShapeDtypeStruct((NC, N_NODES, F), jnp.float32),
            jax.ShapeDtypeStruct((N_NODES, NODE_LEN), jnp.float32),
        ),
        scratch_types=[
            pltpu.VMEM((BASE_CH + 1, CHUNK), jnp.int32),
            pltpu.VMEM((BASE_CH + 1, CHUNK), jnp.int32),
            pltpu.VMEM((RING, CHUNK, F), jnp.float32),
            pltpu.VMEM((ZROWS, F), jnp.float32),
            pltpu.VMEM_SHARED((N_NODES, F), jnp.float32),
            pltpu.SemaphoreType.DMA,
            pltpu.SemaphoreType.DMA,
        ],
    )
    return f(ei3, x)


def _k6_body(h3_ref, wp_ref, wr_ref, b_ref, out_ref, acc_ref):
    i = pl.program_id(0)

    @pl.when(i == 0)
    def _():
        acc_ref[...] = jnp.zeros_like(acc_ref)

    hp = lax.dot_general(h3_ref[...], wp_ref[:, :NODE_LEN],
                         (((1,), (1,)), ((), ())),
                         preferred_element_type=jnp.float32)
    acc_ref[...] += jnp.sum(hp, axis=0, keepdims=True)

    @pl.when(i == pl.num_programs(0) - 1)
    def _():
        g = acc_ref[...] * (1.0 / N_NODES)
        out_ref[...] = (jnp.sum(g * wr_ref[...]) + b_ref[0, 0]).reshape(1, 1)


def _k6(h3, W_pool, W_read, b2):
    blk = 1000
    grid = N_NODES // blk
    return pl.pallas_call(
        _k6_body,
        grid=(grid,),
        in_specs=[
            pl.BlockSpec((blk, NODE_LEN), lambda i: (i, 0)),
            pl.BlockSpec(W_pool.shape, lambda i: (0, 0)),
            pl.BlockSpec(W_read.shape, lambda i: (0, 0)),
            pl.BlockSpec((1, 1), lambda i: (0, 0)),
        ],
        out_specs=pl.BlockSpec((1, 1), lambda i: (0, 0)),
        out_shape=jax.ShapeDtypeStruct((1, 1), jnp.float32),
        scratch_shapes=[pltpu.VMEM((1, GRAPH_LEN), jnp.float32)],
    )(h3, W_pool, W_read, b2)


def kernel(x, edge_index, W_pool, W_read, b_read):
    ei3 = edge_index.astype(jnp.int32).reshape(2, N_CHUNKS, CHUNK)
    _, _, _, h3 = _prop(ei3, x)
    out = _k6(h3, W_pool, W_read, b_read.reshape(1, 1))
    return out.reshape(1)
